# baseline (speedup 1.0000x reference)
import functools

import jax
import jax.numpy as jnp
from jax import lax
from jax.experimental import pallas as pl
from jax.experimental.pallas import tpu as pltpu
from jax.experimental.pallas import tpu_sc as plsc

NU = 20000
NI = 30000
NNODE = NU + NI
D = 64
NEDGE = 800000
NBATCH = 4096

NCORE = 2
NSUB = 16
NW = NCORE * NSUB

EPAD = 819200
E_PER_W = EPAD // NW
E_PER_S = EPAD // NSUB
CHUNK = 1024
STREAM = 128
HHALF = NNODE // NCORE
DUMP = HHALF
ACC_ROWS = HHALF + 8
ROWS_PER_SUB = 1568
SCHUNK = 256

_mesh = plsc.VectorSubcoreMesh(
    core_axis_name="c", subcore_axis_name="s",
    num_cores=NCORE, num_subcores=NSUB)


GCH = 256
GT = E_PER_W // GCH


@functools.partial(
    pl.kernel,
    out_type=jax.ShapeDtypeStruct((EPAD, D), jnp.float32),
    mesh=_mesh,
    compiler_params=pltpu.CompilerParams(
        use_tc_tiling_on_sc=False, needs_layout_passes=False),
    scratch_types=[
        [pltpu.VMEM((GCH // STREAM, STREAM), jnp.int32)] * 2,
        [pltpu.VMEM((GCH, 16), jnp.float32)] * 2,
        [pltpu.VMEM((GCH, D), jnp.float32)] * 2,
        [pltpu.VMEM((GCH, D), jnp.float32)] * 2,
        [pltpu.SemaphoreType.DMA] * 2,
        [pltpu.SemaphoreType.DMA] * 2,
    ],
)
def _sc_gather(ego_hbm, cols_hbm, vals_hbm, s_hbm, ibuf, vbuf, gbuf, obuf,
               gsem, wsem):
    wid = lax.axis_index("s") * NCORE + lax.axis_index("c")
    base = wid * E_PER_W

    def _load_and_fire(t, b):
        e0 = pl.multiple_of(base + t * GCH, GCH)
        r0 = pl.multiple_of(e0 // STREAM, 2)
        pltpu.sync_copy(cols_hbm.at[pl.ds(r0, GCH // STREAM)], ibuf[b])
        pltpu.sync_copy(vals_hbm.at[pl.ds(e0, GCH)], vbuf[b])
        for j in range(GCH // STREAM):
            pltpu.async_copy(
                ego_hbm.at[ibuf[b].at[j]],
                gbuf[b].at[pl.ds(j * STREAM, STREAM)],
                gsem[b],
            )

    def _drain(buf, sem):
        pltpu.make_async_copy(s_hbm.at[pl.ds(0, GCH)], buf, sem).wait()

    _load_and_fire(0, 0)

    @pl.loop(0, GT // 2)
    def _(k):
        for b in range(2):
            t = 2 * k + b
            nb = 1 - b

            @pl.when(t + 1 < GT)
            def _():
                _load_and_fire(t + 1, nb)

            _drain(gbuf[b], gsem[b])

            @pl.when(t >= 2)
            def _():
                _drain(obuf[b], wsem[b])

            @pl.loop(0, GCH, unroll=8)
            def _(e):
                vsp = vbuf[b][e, pl.ds(0, 16)]
                for q in range(D // 16):
                    sl = pl.ds(q * 16, 16)
                    obuf[b][e, sl] = gbuf[b][e, sl] * vsp

            e0 = pl.multiple_of(base + t * GCH, GCH)
            pltpu.async_copy(obuf[b], s_hbm.at[pl.ds(e0, GCH)], wsem[b])

    for b in range(2):
        _drain(obuf[b], wsem[b])


@functools.partial(
    pl.kernel,
    out_type=jax.ShapeDtypeStruct((NNODE, D), jnp.float32),
    mesh=_mesh,
    compiler_params=pltpu.CompilerParams(
        use_tc_tiling_on_sc=False, needs_layout_passes=False),
    scratch_types=[
        pltpu.VMEM((CHUNK // STREAM, STREAM), jnp.int32),
        pltpu.VMEM((CHUNK // STREAM, STREAM), jnp.int32),
        [pltpu.VMEM((STREAM, D), jnp.float32)] * 3,
        pltpu.VMEM_SHARED((ACC_ROWS, D), jnp.float32),
        pltpu.SemaphoreType.DMA,
        [pltpu.SemaphoreType.DMA] * 3,
        [pltpu.SemaphoreType.DMA] * 3,
    ],
)
def _sc_scatter(s_hbm, rows_hbm, side_hbm, rbuf, ibuf, sbuf, acc, sem, ssem,
                lsem):
    c = lax.axis_index("c")
    s = lax.axis_index("s")
    row_base = c * HHALF
    start = pl.multiple_of(
        jnp.minimum(s * ROWS_PER_SUB, HHALF - ROWS_PER_SUB), 8)

    @pl.loop(0, STREAM)
    def _(r):
        for q in range(D // 16):
            sbuf[0][r, pl.ds(q * 16, 16)] = jnp.zeros((16,), jnp.float32)

    zcps = [
        pltpu.async_copy(
            sbuf[0], acc.at[pl.ds(start + k * STREAM, STREAM)], sem)
        for k in range(12)
    ]
    zcps.append(pltpu.async_copy(
        sbuf[0].at[pl.ds(0, 32)],
        acc.at[pl.ds(start + 12 * STREAM, 32)], sem))
    for cp in zcps:
        cp.wait()
    plsc.subcore_barrier()

    ebase = s * E_PER_S
    NGRP = E_PER_S // CHUNK

    def _drain_scatter(b):
        pltpu.make_async_copy(
            s_hbm.at[pl.ds(0, STREAM)], sbuf[b], ssem[b]).wait()

    @pl.loop(0, NGRP)
    def _(g):
        @pl.when(g > 0)
        def _():
            for b in range(3):
                _drain_scatter(b)

        e0 = pl.multiple_of(ebase + g * CHUNK, CHUNK)
        r0 = pl.multiple_of(e0 // STREAM, 8)
        pltpu.sync_copy(rows_hbm.at[pl.ds(r0, CHUNK // STREAM)], rbuf)
        for j in range(CHUNK // STREAM):
            for q in range(STREAM // 16):
                r = rbuf[j, pl.ds(q * 16, 16)]
                loc = r - row_base
                ok = (loc >= 0) & (loc < HHALF)
                ibuf[j, pl.ds(q * 16, 16)] = jnp.where(ok, loc, DUMP)
        for p in range(2):
            pltpu.async_copy(
                s_hbm.at[pl.ds(e0 + p * STREAM, STREAM)], sbuf[p], lsem[p])
        for u in range(CHUNK // STREAM):
            b = u % 3
            if u + 2 < CHUNK // STREAM:
                fb = (u + 2) % 3
                if u + 2 >= 3:
                    _drain_scatter(fb)
                pltpu.async_copy(
                    s_hbm.at[pl.ds(e0 + (u + 2) * STREAM, STREAM)],
                    sbuf[fb], lsem[fb])
            pltpu.make_async_copy(
                s_hbm.at[pl.ds(0, STREAM)], sbuf[b], lsem[b]).wait()
            pltpu.async_copy(sbuf[b], acc.at[ibuf.at[u]], ssem[b], add=True)

    for b in range(3):
        _drain_scatter(b)
    plsc.subcore_barrier()
    pltpu.async_copy(
        acc.at[pl.ds(start, ROWS_PER_SUB)],
        side_hbm.at[pl.ds(pl.multiple_of(row_base + start, 8), ROWS_PER_SUB)],
        sem,
    ).wait()


N_IDX = 3 * NBATCH
IDX_PER_W = 1024
GCHUNK = 256


@functools.partial(
    pl.kernel,
    out_type=jax.ShapeDtypeStruct((N_IDX, 4 * D), jnp.float32),
    mesh=_mesh,
    compiler_params=pltpu.CompilerParams(
        use_tc_tiling_on_sc=False, needs_layout_passes=False),
    scratch_types=[
        pltpu.VMEM((IDX_PER_W // STREAM, STREAM), jnp.int32),
        [pltpu.VMEM((STREAM, 4 * D), jnp.float32)] * 3,
        [pltpu.SemaphoreType.DMA] * 3,
        [pltpu.SemaphoreType.DMA] * 3,
    ],
)
def _sc_batch_gather(ae_hbm, idx_hbm, out_hbm, idx_v, bufs, gsem, wsem):
    wid = lax.axis_index("s") * NCORE + lax.axis_index("c")
    nst = IDX_PER_W // STREAM

    @pl.when(wid < N_IDX // IDX_PER_W)
    def _():
        base = pl.multiple_of(wid * IDX_PER_W, IDX_PER_W)
        r0 = pl.multiple_of(base // STREAM, 8)
        pltpu.sync_copy(idx_hbm.at[pl.ds(r0, nst)], idx_v)
        for st in range(min(3, nst)):
            pltpu.async_copy(
                ae_hbm.at[idx_v.at[st]], bufs[st], gsem[st])
        for st in range(nst):
            b = st % 3
            pltpu.make_async_copy(
                out_hbm.at[pl.ds(0, STREAM)], bufs[b], gsem[b]).wait()
            pltpu.async_copy(
                bufs[b], out_hbm.at[pl.ds(base + st * STREAM, STREAM)],
                wsem[b])
            if st + 3 < nst:
                pltpu.make_async_copy(
                    bufs[b], out_hbm.at[pl.ds(0, STREAM)], wsem[b]).wait()
                pltpu.async_copy(
                    ae_hbm.at[idx_v.at[st + 3]], bufs[b], gsem[b])
        for st in range(nst - 3, nst):
            b = st % 3
            pltpu.make_async_copy(
                bufs[b], out_hbm.at[pl.ds(0, STREAM)], wsem[b]).wait()


def _vrep_body(v_ref, o_ref):
    o_ref[...] = jnp.broadcast_to(v_ref[...], o_ref.shape)


def _tc_vrep(vals2d):
    blk = 8192
    return pl.pallas_call(
        _vrep_body,
        grid=(EPAD // blk,),
        in_specs=[pl.BlockSpec((blk, 1), lambda i: (i, 0))],
        out_specs=pl.BlockSpec((blk, 16), lambda i: (i, 0)),
        out_shape=jax.ShapeDtypeStruct((EPAD, 16), jnp.float32),
    )(vals2d)


def _dense_body(side_ref, ego_ref, wg_ref, bg_ref, wb_ref, bb_ref,
                nxt_ref, nrm_ref):
    side = side_ref[...]
    ego = ego_ref[...]
    sum_emb = (
        jnp.dot(side, wg_ref[...], preferred_element_type=jnp.float32,
                precision=lax.Precision.HIGHEST)
        + bg_ref[...]
    )
    bi_emb = jnp.dot(ego * side, wb_ref[...] + bb_ref[...],
                     preferred_element_type=jnp.float32,
                     precision=lax.Precision.HIGHEST)
    h = sum_emb + bi_emb
    h = jnp.where(h >= 0, h, 0.2 * h)
    nxt_ref[...] = h
    nrm = jnp.sqrt(jnp.sum(h * h, axis=1, keepdims=True))
    nrm_ref[...] = h / jnp.maximum(nrm, 1e-12)


def _tc_dense(side, ego, wg, bg, wb, bb):
    blk = 400
    return pl.pallas_call(
        _dense_body,
        grid=(NNODE // blk,),
        in_specs=[
            pl.BlockSpec((blk, D), lambda i: (i, 0)),
            pl.BlockSpec((blk, D), lambda i: (i, 0)),
            pl.BlockSpec((D, D), lambda i: (0, 0)),
            pl.BlockSpec((1, D), lambda i: (0, 0)),
            pl.BlockSpec((D, D), lambda i: (0, 0)),
            pl.BlockSpec((1, D), lambda i: (0, 0)),
        ],
        out_specs=[
            pl.BlockSpec((blk, D), lambda i: (i, 0)),
            pl.BlockSpec((blk, D), lambda i: (i, 0)),
        ],
        out_shape=[
            jax.ShapeDtypeStruct((NNODE, D), jnp.float32),
            jax.ShapeDtypeStruct((NNODE, D), jnp.float32),
        ],
    )(side, ego, wg, bg, wb, bb)


def _logits_body(u_ref, p_ref, n_ref, lp_ref, ln_ref, pp_ref, pn_ref):
    u = u_ref[...]
    lp = jnp.sum(u * p_ref[...], axis=1, keepdims=True)
    ln = jnp.sum(u * n_ref[...], axis=1, keepdims=True)
    lp_ref[...] = lp
    ln_ref[...] = ln
    pp_ref[...] = jax.nn.sigmoid(lp)
    pn_ref[...] = jax.nn.sigmoid(ln)


def _tc_logits(u, p, n):
    blk = 512
    return pl.pallas_call(
        _logits_body,
        grid=(NBATCH // blk,),
        in_specs=[
            pl.BlockSpec((blk, 4 * D), lambda i: (i, 0)),
            pl.BlockSpec((blk, 4 * D), lambda i: (i, 0)),
            pl.BlockSpec((blk, 4 * D), lambda i: (i, 0)),
        ],
        out_specs=[pl.BlockSpec((blk, 1), lambda i: (i, 0))] * 4,
        out_shape=[jax.ShapeDtypeStruct((NBATCH, 1), jnp.float32)] * 4,
    )(u, p, n)


def kernel(users, pos_items, neg_items, adj_rows, adj_cols, adj_vals,
           user_emb, item_emb,
           W_gc_0, b_gc_0, W_bi_0, b_bi_0,
           W_gc_1, b_gc_1, W_bi_1, b_bi_1,
           W_gc_2, b_gc_2, W_bi_2, b_bi_2):
    Wg = [W_gc_0, W_gc_1, W_gc_2]
    bg = [b_gc_0, b_gc_1, b_gc_2]
    Wb = [W_bi_0, W_bi_1, W_bi_2]
    bb = [b_bi_0, b_bi_1, b_bi_2]

    pad = EPAD - NEDGE
    cols2d = jnp.pad(adj_cols.astype(jnp.int32), (0, pad)).reshape(
        EPAD // STREAM, STREAM)
    rows2d = jnp.pad(adj_rows.astype(jnp.int32), (0, pad)).reshape(
        EPAD // STREAM, STREAM)
    vrep = _tc_vrep(jnp.pad(adj_vals, (0, pad)).reshape(EPAD, 1))

    ego = jnp.concatenate([user_emb, item_emb], axis=0)
    norms = [ego]
    for k in range(3):
        s = _sc_gather(ego, cols2d, vrep)
        side = _sc_scatter(s, rows2d)
        ego, nrm = _tc_dense(side, ego, Wg[k], bg[k], Wb[k], bb[k])
        norms.append(nrm)

    ae = jnp.concatenate(norms, axis=1)

    idx = jnp.concatenate([
        users.astype(jnp.int32),
        NU + pos_items.astype(jnp.int32),
        NU + neg_items.astype(jnp.int32),
    ]).reshape(N_IDX // STREAM, STREAM)
    picked = _sc_batch_gather(ae, idx)
    u_out = picked[:NBATCH]
    pos_i = picked[NBATCH:2 * NBATCH]
    neg_i = picked[2 * NBATCH:]

    lp, ln, pp, pn = _tc_logits(u_out, pos_i, neg_i)
    logits = jnp.concatenate([lp, ln], axis=0)
    prediction = jnp.concatenate([pp, pn], axis=0)
    i_sel = jnp.concatenate([pos_i, neg_i], axis=0)

    return (ae, u_out, i_sel, pos_i, neg_i, logits, prediction)


# speedup vs baseline: 1.1578x; 1.1578x over previous
import functools

import jax
import jax.numpy as jnp
from jax import lax
from jax.experimental import pallas as pl
from jax.experimental.pallas import tpu as pltpu
from jax.experimental.pallas import tpu_sc as plsc

NU = 20000
NI = 30000
NNODE = NU + NI
D = 64
NEDGE = 800000
NBATCH = 4096

NCORE = 2
NSUB = 16
NW = NCORE * NSUB

EPAD = 819200
E_PER_W = EPAD // NW
E_PER_S = EPAD // NSUB
CHUNK = 1024
STREAM = 128
HHALF = NNODE // NCORE
DUMP = HHALF
ACC_ROWS = HHALF + 8
ROWS_PER_SUB = 1568
SCHUNK = 256

_mesh = plsc.VectorSubcoreMesh(
    core_axis_name="c", subcore_axis_name="s",
    num_cores=NCORE, num_subcores=NSUB)


GCH = 256
GT = E_PER_W // GCH


@functools.partial(
    pl.kernel,
    out_type=jax.ShapeDtypeStruct((EPAD, D), jnp.float32),
    mesh=_mesh,
    compiler_params=pltpu.CompilerParams(
        use_tc_tiling_on_sc=False, needs_layout_passes=False),
    scratch_types=[
        [pltpu.VMEM((GCH // STREAM, STREAM), jnp.int32)] * 4,
        [pltpu.VMEM((GCH,), jnp.float32)] * 4,
        [pltpu.VMEM((GCH, D), jnp.float32)] * 4,
        [pltpu.VMEM((GCH, D), jnp.float32)] * 2,
        [pltpu.SemaphoreType.DMA] * 4,
        [pltpu.SemaphoreType.DMA] * 2,
    ],
)
def _sc_gather(ego_hbm, cols_hbm, vals_hbm, s_hbm, ibuf, vbuf, gbuf, obuf,
               gsem, wsem):
    wid = lax.axis_index("s") * NCORE + lax.axis_index("c")
    base = wid * E_PER_W

    def _load_and_fire(t, b):
        e0 = pl.multiple_of(base + t * GCH, GCH)
        r0 = pl.multiple_of(e0 // STREAM, 2)
        pltpu.sync_copy(cols_hbm.at[pl.ds(r0, GCH // STREAM)], ibuf[b])
        pltpu.sync_copy(vals_hbm.at[pl.ds(e0, GCH)], vbuf[b])
        for j in range(GCH // STREAM):
            pltpu.async_copy(
                ego_hbm.at[ibuf[b].at[j]],
                gbuf[b].at[pl.ds(j * STREAM, STREAM)],
                gsem[b],
            )

    def _drain(buf, sem):
        pltpu.make_async_copy(s_hbm.at[pl.ds(0, GCH)], buf, sem).wait()

    for p in range(3):
        _load_and_fire(p, p)

    @pl.loop(0, GT // 4)
    def _(k):
        for u in range(4):
            t = 4 * k + u
            ob = u % 2

            @pl.when(t + 3 < GT)
            def _():
                _load_and_fire(t + 3, (u + 3) % 4)

            _drain(gbuf[u], gsem[u])

            @pl.when(t >= 2)
            def _():
                _drain(obuf[ob], wsem[ob])

            @pl.loop(0, GCH, unroll=8)
            def _(e):
                iv = jnp.full((16,), e, jnp.int32)
                vsp = plsc.load_gather(vbuf[u], [iv])
                for q in range(D // 16):
                    sl = pl.ds(q * 16, 16)
                    obuf[ob][e, sl] = gbuf[u][e, sl] * vsp

            e0 = pl.multiple_of(base + t * GCH, GCH)
            pltpu.async_copy(obuf[ob], s_hbm.at[pl.ds(e0, GCH)], wsem[ob])

    for b in range(2):
        _drain(obuf[b], wsem[b])


@functools.partial(
    pl.kernel,
    out_type=jax.ShapeDtypeStruct((NNODE, D), jnp.float32),
    mesh=_mesh,
    compiler_params=pltpu.CompilerParams(
        use_tc_tiling_on_sc=False, needs_layout_passes=False),
    scratch_types=[
        pltpu.VMEM((CHUNK // STREAM, STREAM), jnp.int32),
        pltpu.VMEM((CHUNK // STREAM, STREAM), jnp.int32),
        [pltpu.VMEM((STREAM, D), jnp.float32)] * 3,
        pltpu.VMEM_SHARED((ACC_ROWS, D), jnp.float32),
        pltpu.SemaphoreType.DMA,
        [pltpu.SemaphoreType.DMA] * 3,
        [pltpu.SemaphoreType.DMA] * 3,
    ],
)
def _sc_scatter(s_hbm, rows_hbm, side_hbm, rbuf, ibuf, sbuf, acc, sem, ssem,
                lsem):
    c = lax.axis_index("c")
    s = lax.axis_index("s")
    row_base = c * HHALF
    start = pl.multiple_of(
        jnp.minimum(s * ROWS_PER_SUB, HHALF - ROWS_PER_SUB), 8)

    @pl.loop(0, STREAM)
    def _(r):
        for q in range(D // 16):
            sbuf[0][r, pl.ds(q * 16, 16)] = jnp.zeros((16,), jnp.float32)

    zcps = [
        pltpu.async_copy(
            sbuf[0], acc.at[pl.ds(start + k * STREAM, STREAM)], sem)
        for k in range(12)
    ]
    zcps.append(pltpu.async_copy(
        sbuf[0].at[pl.ds(0, 32)],
        acc.at[pl.ds(start + 12 * STREAM, 32)], sem))
    for cp in zcps:
        cp.wait()
    plsc.subcore_barrier()

    ebase = s * E_PER_S
    NGRP = E_PER_S // CHUNK

    def _drain_scatter(b):
        pltpu.make_async_copy(
            s_hbm.at[pl.ds(0, STREAM)], sbuf[b], ssem[b]).wait()

    @pl.loop(0, NGRP)
    def _(g):
        @pl.when(g > 0)
        def _():
            for b in range(3):
                _drain_scatter(b)

        e0 = pl.multiple_of(ebase + g * CHUNK, CHUNK)
        r0 = pl.multiple_of(e0 // STREAM, 8)
        pltpu.sync_copy(rows_hbm.at[pl.ds(r0, CHUNK // STREAM)], rbuf)
        for j in range(CHUNK // STREAM):
            for q in range(STREAM // 16):
                r = rbuf[j, pl.ds(q * 16, 16)]
                loc = r - row_base
                ok = (loc >= 0) & (loc < HHALF)
                ibuf[j, pl.ds(q * 16, 16)] = jnp.where(ok, loc, DUMP)
        for p in range(2):
            pltpu.async_copy(
                s_hbm.at[pl.ds(e0 + p * STREAM, STREAM)], sbuf[p], lsem[p])
        for u in range(CHUNK // STREAM):
            b = u % 3
            if u + 2 < CHUNK // STREAM:
                fb = (u + 2) % 3
                if u + 2 >= 3:
                    _drain_scatter(fb)
                pltpu.async_copy(
                    s_hbm.at[pl.ds(e0 + (u + 2) * STREAM, STREAM)],
                    sbuf[fb], lsem[fb])
            pltpu.make_async_copy(
                s_hbm.at[pl.ds(0, STREAM)], sbuf[b], lsem[b]).wait()
            pltpu.async_copy(sbuf[b], acc.at[ibuf.at[u]], ssem[b], add=True)

    for b in range(3):
        _drain_scatter(b)
    plsc.subcore_barrier()
    pltpu.async_copy(
        acc.at[pl.ds(start, ROWS_PER_SUB)],
        side_hbm.at[pl.ds(pl.multiple_of(row_base + start, 8), ROWS_PER_SUB)],
        sem,
    ).wait()


N_IDX = 3 * NBATCH
IDX_PER_W = 1024
GCHUNK = 256


@functools.partial(
    pl.kernel,
    out_type=jax.ShapeDtypeStruct((N_IDX, 4 * D), jnp.float32),
    mesh=_mesh,
    compiler_params=pltpu.CompilerParams(
        use_tc_tiling_on_sc=False, needs_layout_passes=False),
    scratch_types=[
        pltpu.VMEM((IDX_PER_W // STREAM, STREAM), jnp.int32),
        [pltpu.VMEM((STREAM, 4 * D), jnp.float32)] * 3,
        [pltpu.SemaphoreType.DMA] * 3,
        [pltpu.SemaphoreType.DMA] * 3,
    ],
)
def _sc_batch_gather(ae_hbm, idx_hbm, out_hbm, idx_v, bufs, gsem, wsem):
    wid = lax.axis_index("s") * NCORE + lax.axis_index("c")
    nst = IDX_PER_W // STREAM

    @pl.when(wid < N_IDX // IDX_PER_W)
    def _():
        base = pl.multiple_of(wid * IDX_PER_W, IDX_PER_W)
        r0 = pl.multiple_of(base // STREAM, 8)
        pltpu.sync_copy(idx_hbm.at[pl.ds(r0, nst)], idx_v)
        for st in range(min(3, nst)):
            pltpu.async_copy(
                ae_hbm.at[idx_v.at[st]], bufs[st], gsem[st])
        for st in range(nst):
            b = st % 3
            pltpu.make_async_copy(
                out_hbm.at[pl.ds(0, STREAM)], bufs[b], gsem[b]).wait()
            pltpu.async_copy(
                bufs[b], out_hbm.at[pl.ds(base + st * STREAM, STREAM)],
                wsem[b])
            if st + 3 < nst:
                pltpu.make_async_copy(
                    bufs[b], out_hbm.at[pl.ds(0, STREAM)], wsem[b]).wait()
                pltpu.async_copy(
                    ae_hbm.at[idx_v.at[st + 3]], bufs[b], gsem[b])
        for st in range(nst - 3, nst):
            b = st % 3
            pltpu.make_async_copy(
                bufs[b], out_hbm.at[pl.ds(0, STREAM)], wsem[b]).wait()


def _dense_body(side_ref, ego_ref, wg_ref, bg_ref, wb_ref, bb_ref,
                nxt_ref, nrm_ref):
    side = side_ref[...]
    ego = ego_ref[...]
    sum_emb = (
        jnp.dot(side, wg_ref[...], preferred_element_type=jnp.float32,
                precision=lax.Precision.HIGHEST)
        + bg_ref[...]
    )
    bi_emb = jnp.dot(ego * side, wb_ref[...] + bb_ref[...],
                     preferred_element_type=jnp.float32,
                     precision=lax.Precision.HIGHEST)
    h = sum_emb + bi_emb
    h = jnp.where(h >= 0, h, 0.2 * h)
    nxt_ref[...] = h
    nrm = jnp.sqrt(jnp.sum(h * h, axis=1, keepdims=True))
    nrm_ref[...] = h / jnp.maximum(nrm, 1e-12)


def _tc_dense(side, ego, wg, bg, wb, bb):
    blk = 400
    return pl.pallas_call(
        _dense_body,
        grid=(NNODE // blk,),
        in_specs=[
            pl.BlockSpec((blk, D), lambda i: (i, 0)),
            pl.BlockSpec((blk, D), lambda i: (i, 0)),
            pl.BlockSpec((D, D), lambda i: (0, 0)),
            pl.BlockSpec((1, D), lambda i: (0, 0)),
            pl.BlockSpec((D, D), lambda i: (0, 0)),
            pl.BlockSpec((1, D), lambda i: (0, 0)),
        ],
        out_specs=[
            pl.BlockSpec((blk, D), lambda i: (i, 0)),
            pl.BlockSpec((blk, D), lambda i: (i, 0)),
        ],
        out_shape=[
            jax.ShapeDtypeStruct((NNODE, D), jnp.float32),
            jax.ShapeDtypeStruct((NNODE, D), jnp.float32),
        ],
    )(side, ego, wg, bg, wb, bb)


def _logits_body(u_ref, p_ref, n_ref, lp_ref, ln_ref, pp_ref, pn_ref):
    u = u_ref[...]
    lp = jnp.sum(u * p_ref[...], axis=1, keepdims=True)
    ln = jnp.sum(u * n_ref[...], axis=1, keepdims=True)
    lp_ref[...] = lp
    ln_ref[...] = ln
    pp_ref[...] = jax.nn.sigmoid(lp)
    pn_ref[...] = jax.nn.sigmoid(ln)


def _tc_logits(u, p, n):
    blk = 512
    return pl.pallas_call(
        _logits_body,
        grid=(NBATCH // blk,),
        in_specs=[
            pl.BlockSpec((blk, 4 * D), lambda i: (i, 0)),
            pl.BlockSpec((blk, 4 * D), lambda i: (i, 0)),
            pl.BlockSpec((blk, 4 * D), lambda i: (i, 0)),
        ],
        out_specs=[pl.BlockSpec((blk, 1), lambda i: (i, 0))] * 4,
        out_shape=[jax.ShapeDtypeStruct((NBATCH, 1), jnp.float32)] * 4,
    )(u, p, n)


def kernel(users, pos_items, neg_items, adj_rows, adj_cols, adj_vals,
           user_emb, item_emb,
           W_gc_0, b_gc_0, W_bi_0, b_bi_0,
           W_gc_1, b_gc_1, W_bi_1, b_bi_1,
           W_gc_2, b_gc_2, W_bi_2, b_bi_2):
    Wg = [W_gc_0, W_gc_1, W_gc_2]
    bg = [b_gc_0, b_gc_1, b_gc_2]
    Wb = [W_bi_0, W_bi_1, W_bi_2]
    bb = [b_bi_0, b_bi_1, b_bi_2]

    pad = EPAD - NEDGE
    cols2d = jnp.pad(adj_cols.astype(jnp.int32), (0, pad)).reshape(
        EPAD // STREAM, STREAM)
    rows2d = jnp.pad(adj_rows.astype(jnp.int32), (0, pad)).reshape(
        EPAD // STREAM, STREAM)
    vals1d = jnp.pad(adj_vals, (0, pad))

    ego = jnp.concatenate([user_emb, item_emb], axis=0)
    norms = [ego]
    for k in range(3):
        s = _sc_gather(ego, cols2d, vals1d)
        side = _sc_scatter(s, rows2d)
        ego, nrm = _tc_dense(side, ego, Wg[k], bg[k], Wb[k], bb[k])
        norms.append(nrm)

    ae = jnp.concatenate(norms, axis=1)

    idx = jnp.concatenate([
        users.astype(jnp.int32),
        NU + pos_items.astype(jnp.int32),
        NU + neg_items.astype(jnp.int32),
    ]).reshape(N_IDX // STREAM, STREAM)
    picked = _sc_batch_gather(ae, idx)
    u_out = picked[:NBATCH]
    pos_i = picked[NBATCH:2 * NBATCH]
    neg_i = picked[2 * NBATCH:]

    lp, ln, pp, pn = _tc_logits(u_out, pos_i, neg_i)
    logits = jnp.concatenate([lp, ln], axis=0)
    prediction = jnp.concatenate([pp, pn], axis=0)
    i_sel = jnp.concatenate([pos_i, neg_i], axis=0)

    return (ae, u_out, i_sel, pos_i, neg_i, logits, prediction)


# speedup vs baseline: 1.1693x; 1.0100x over previous
import functools

import jax
import jax.numpy as jnp
from jax import lax
from jax.experimental import pallas as pl
from jax.experimental.pallas import tpu as pltpu
from jax.experimental.pallas import tpu_sc as plsc

NU = 20000
NI = 30000
NNODE = NU + NI
D = 64
NEDGE = 800000
NBATCH = 4096

NCORE = 2
NSUB = 16
NW = NCORE * NSUB

EPAD = 819200
E_PER_W = EPAD // NW
E_PER_S = EPAD // NSUB
CHUNK = 1024
STREAM = 128
HHALF = NNODE // NCORE
DUMP = HHALF
ACC_ROWS = HHALF + 8
ROWS_PER_SUB = 1568
SCHUNK = 256

_mesh = plsc.VectorSubcoreMesh(
    core_axis_name="c", subcore_axis_name="s",
    num_cores=NCORE, num_subcores=NSUB)


GCH = 256
GT = E_PER_W // GCH


@functools.partial(
    pl.kernel,
    out_type=jax.ShapeDtypeStruct((EPAD, D), jnp.float32),
    mesh=_mesh,
    compiler_params=pltpu.CompilerParams(
        use_tc_tiling_on_sc=False, needs_layout_passes=False),
    scratch_types=[
        [pltpu.VMEM((GCH // STREAM, STREAM), jnp.int32)] * 4,
        [pltpu.VMEM((GCH,), jnp.float32)] * 4,
        [pltpu.VMEM((GCH, D), jnp.float32)] * 4,
        [pltpu.VMEM((GCH, D), jnp.float32)] * 2,
        [pltpu.SemaphoreType.DMA] * 4,
        [pltpu.SemaphoreType.DMA] * 2,
    ],
)
def _sc_gather(ego_hbm, cols_hbm, vals_hbm, s_hbm, ibuf, vbuf, gbuf, obuf,
               gsem, wsem):
    wid = lax.axis_index("s") * NCORE + lax.axis_index("c")
    base = wid * E_PER_W

    def _load_and_fire(t, b):
        e0 = pl.multiple_of(base + t * GCH, GCH)
        r0 = pl.multiple_of(e0 // STREAM, 2)
        pltpu.sync_copy(cols_hbm.at[pl.ds(r0, GCH // STREAM)], ibuf[b])
        pltpu.sync_copy(vals_hbm.at[pl.ds(e0, GCH)], vbuf[b])
        for j in range(GCH // STREAM):
            pltpu.async_copy(
                ego_hbm.at[ibuf[b].at[j]],
                gbuf[b].at[pl.ds(j * STREAM, STREAM)],
                gsem[b],
            )

    def _drain(buf, sem):
        pltpu.make_async_copy(s_hbm.at[pl.ds(0, GCH)], buf, sem).wait()

    for p in range(3):
        _load_and_fire(p, p)

    @pl.loop(0, GT // 4)
    def _(k):
        for u in range(4):
            t = 4 * k + u
            ob = u % 2

            @pl.when(t + 3 < GT)
            def _():
                _load_and_fire(t + 3, (u + 3) % 4)

            _drain(gbuf[u], gsem[u])

            @pl.when(t >= 2)
            def _():
                _drain(obuf[ob], wsem[ob])

            @pl.loop(0, GCH, unroll=8)
            def _(e):
                iv = jnp.full((16,), e, jnp.int32)
                vsp = plsc.load_gather(vbuf[u], [iv])
                for q in range(D // 16):
                    sl = pl.ds(q * 16, 16)
                    obuf[ob][e, sl] = gbuf[u][e, sl] * vsp

            e0 = pl.multiple_of(base + t * GCH, GCH)
            pltpu.async_copy(obuf[ob], s_hbm.at[pl.ds(e0, GCH)], wsem[ob])

    for b in range(2):
        _drain(obuf[b], wsem[b])


@functools.partial(
    pl.kernel,
    out_type=jax.ShapeDtypeStruct((NNODE, D), jnp.float32),
    mesh=_mesh,
    compiler_params=pltpu.CompilerParams(
        use_tc_tiling_on_sc=False, needs_layout_passes=False),
    scratch_types=[
        pltpu.VMEM((CHUNK // STREAM, STREAM), jnp.int32),
        pltpu.VMEM((CHUNK // STREAM, STREAM), jnp.int32),
        [pltpu.VMEM((STREAM, D), jnp.float32)] * 3,
        pltpu.VMEM_SHARED((ACC_ROWS, D), jnp.float32),
        pltpu.SemaphoreType.DMA,
        [pltpu.SemaphoreType.DMA] * 3,
        [pltpu.SemaphoreType.DMA] * 3,
    ],
)
def _sc_scatter(s_hbm, rows_hbm, side_hbm, rbuf, ibuf, sbuf, acc, sem, ssem,
                lsem):
    c = lax.axis_index("c")
    s = lax.axis_index("s")
    row_base = c * HHALF
    start = pl.multiple_of(
        jnp.minimum(s * ROWS_PER_SUB, HHALF - ROWS_PER_SUB), 8)

    @pl.loop(0, STREAM)
    def _(r):
        for q in range(D // 16):
            sbuf[0][r, pl.ds(q * 16, 16)] = jnp.zeros((16,), jnp.float32)

    zcps = [
        pltpu.async_copy(
            sbuf[0], acc.at[pl.ds(start + k * STREAM, STREAM)], sem)
        for k in range(12)
    ]
    zcps.append(pltpu.async_copy(
        sbuf[0].at[pl.ds(0, 32)],
        acc.at[pl.ds(start + 12 * STREAM, 32)], sem))
    for cp in zcps:
        cp.wait()
    plsc.subcore_barrier()

    ebase = s * E_PER_S
    NGRP = E_PER_S // CHUNK

    def _drain_scatter(b):
        pltpu.make_async_copy(
            s_hbm.at[pl.ds(0, STREAM)], sbuf[b], ssem[b]).wait()

    @pl.loop(0, NGRP)
    def _(g):
        @pl.when(g > 0)
        def _():
            for b in range(3):
                _drain_scatter(b)

        e0 = pl.multiple_of(ebase + g * CHUNK, CHUNK)
        r0 = pl.multiple_of(e0 // STREAM, 8)
        pltpu.sync_copy(rows_hbm.at[pl.ds(r0, CHUNK // STREAM)], rbuf)
        for j in range(CHUNK // STREAM):
            for q in range(STREAM // 16):
                r = rbuf[j, pl.ds(q * 16, 16)]
                loc = r - row_base
                ok = (loc >= 0) & (loc < HHALF)
                ibuf[j, pl.ds(q * 16, 16)] = jnp.where(ok, loc, DUMP)
        for p in range(2):
            pltpu.async_copy(
                s_hbm.at[pl.ds(e0 + p * STREAM, STREAM)], sbuf[p], lsem[p])
        for u in range(CHUNK // STREAM):
            b = u % 3
            if u + 2 < CHUNK // STREAM:
                fb = (u + 2) % 3
                if u + 2 >= 3:
                    _drain_scatter(fb)
                pltpu.async_copy(
                    s_hbm.at[pl.ds(e0 + (u + 2) * STREAM, STREAM)],
                    sbuf[fb], lsem[fb])
            pltpu.make_async_copy(
                s_hbm.at[pl.ds(0, STREAM)], sbuf[b], lsem[b]).wait()
            pltpu.async_copy(sbuf[b], acc.at[ibuf.at[u]], ssem[b], add=True)

    for b in range(3):
        _drain_scatter(b)
    plsc.subcore_barrier()
    pltpu.async_copy(
        acc.at[pl.ds(start, ROWS_PER_SUB)],
        side_hbm.at[pl.ds(pl.multiple_of(row_base + start, 8), ROWS_PER_SUB)],
        sem,
    ).wait()


N_IDX = 3 * NBATCH
IDX_PER_W = 512
GCHUNK = 256


@functools.partial(
    pl.kernel,
    out_type=jax.ShapeDtypeStruct((N_IDX, 4 * D), jnp.float32),
    mesh=_mesh,
    compiler_params=pltpu.CompilerParams(
        use_tc_tiling_on_sc=False, needs_layout_passes=False),
    scratch_types=[
        pltpu.VMEM((IDX_PER_W // STREAM, STREAM), jnp.int32),
        [pltpu.VMEM((STREAM, 4 * D), jnp.float32)] * 3,
        [pltpu.SemaphoreType.DMA] * 3,
        [pltpu.SemaphoreType.DMA] * 3,
    ],
)
def _sc_batch_gather(ae_hbm, idx_hbm, out_hbm, idx_v, bufs, gsem, wsem):
    wid = lax.axis_index("s") * NCORE + lax.axis_index("c")
    nst = IDX_PER_W // STREAM

    @pl.when(wid < N_IDX // IDX_PER_W)
    def _():
        base = pl.multiple_of(wid * IDX_PER_W, IDX_PER_W)
        r0 = pl.multiple_of(base // STREAM, 4)
        pltpu.sync_copy(idx_hbm.at[pl.ds(r0, nst)], idx_v)
        for st in range(min(3, nst)):
            pltpu.async_copy(
                ae_hbm.at[idx_v.at[st]], bufs[st], gsem[st])
        for st in range(nst):
            b = st % 3
            pltpu.make_async_copy(
                out_hbm.at[pl.ds(0, STREAM)], bufs[b], gsem[b]).wait()
            pltpu.async_copy(
                bufs[b], out_hbm.at[pl.ds(base + st * STREAM, STREAM)],
                wsem[b])
            if st + 3 < nst:
                pltpu.make_async_copy(
                    bufs[b], out_hbm.at[pl.ds(0, STREAM)], wsem[b]).wait()
                pltpu.async_copy(
                    ae_hbm.at[idx_v.at[st + 3]], bufs[b], gsem[b])
        for st in range(nst - 3, nst):
            b = st % 3
            pltpu.make_async_copy(
                bufs[b], out_hbm.at[pl.ds(0, STREAM)], wsem[b]).wait()


def _dense_body(side_ref, ego_ref, wg_ref, bg_ref, wb_ref, bb_ref,
                nxt_ref, nrm_ref):
    side = side_ref[...]
    ego = ego_ref[...]
    sum_emb = (
        jnp.dot(side, wg_ref[...], preferred_element_type=jnp.float32,
                precision=lax.Precision.HIGHEST)
        + bg_ref[...]
    )
    bi_emb = jnp.dot(ego * side, wb_ref[...] + bb_ref[...],
                     preferred_element_type=jnp.float32,
                     precision=lax.Precision.HIGHEST)
    h = sum_emb + bi_emb
    h = jnp.where(h >= 0, h, 0.2 * h)
    nxt_ref[...] = h
    nrm = jnp.sqrt(jnp.sum(h * h, axis=1, keepdims=True))
    nrm_ref[...] = h / jnp.maximum(nrm, 1e-12)


def _tc_dense(side, ego, wg, bg, wb, bb):
    blk = 400
    return pl.pallas_call(
        _dense_body,
        grid=(NNODE // blk,),
        in_specs=[
            pl.BlockSpec((blk, D), lambda i: (i, 0)),
            pl.BlockSpec((blk, D), lambda i: (i, 0)),
            pl.BlockSpec((D, D), lambda i: (0, 0)),
            pl.BlockSpec((1, D), lambda i: (0, 0)),
            pl.BlockSpec((D, D), lambda i: (0, 0)),
            pl.BlockSpec((1, D), lambda i: (0, 0)),
        ],
        out_specs=[
            pl.BlockSpec((blk, D), lambda i: (i, 0)),
            pl.BlockSpec((blk, D), lambda i: (i, 0)),
        ],
        out_shape=[
            jax.ShapeDtypeStruct((NNODE, D), jnp.float32),
            jax.ShapeDtypeStruct((NNODE, D), jnp.float32),
        ],
    )(side, ego, wg, bg, wb, bb)


def _logits_body(u_ref, p_ref, n_ref, lp_ref, ln_ref, pp_ref, pn_ref):
    u = u_ref[...]
    lp = jnp.sum(u * p_ref[...], axis=1, keepdims=True)
    ln = jnp.sum(u * n_ref[...], axis=1, keepdims=True)
    lp_ref[...] = lp
    ln_ref[...] = ln
    pp_ref[...] = jax.nn.sigmoid(lp)
    pn_ref[...] = jax.nn.sigmoid(ln)


def _tc_logits(u, p, n):
    blk = 512
    return pl.pallas_call(
        _logits_body,
        grid=(NBATCH // blk,),
        in_specs=[
            pl.BlockSpec((blk, 4 * D), lambda i: (i, 0)),
            pl.BlockSpec((blk, 4 * D), lambda i: (i, 0)),
            pl.BlockSpec((blk, 4 * D), lambda i: (i, 0)),
        ],
        out_specs=[pl.BlockSpec((blk, 1), lambda i: (i, 0))] * 4,
        out_shape=[jax.ShapeDtypeStruct((NBATCH, 1), jnp.float32)] * 4,
    )(u, p, n)


def kernel(users, pos_items, neg_items, adj_rows, adj_cols, adj_vals,
           user_emb, item_emb,
           W_gc_0, b_gc_0, W_bi_0, b_bi_0,
           W_gc_1, b_gc_1, W_bi_1, b_bi_1,
           W_gc_2, b_gc_2, W_bi_2, b_bi_2):
    Wg = [W_gc_0, W_gc_1, W_gc_2]
    bg = [b_gc_0, b_gc_1, b_gc_2]
    Wb = [W_bi_0, W_bi_1, W_bi_2]
    bb = [b_bi_0, b_bi_1, b_bi_2]

    pad = EPAD - NEDGE
    cols2d = jnp.pad(adj_cols.astype(jnp.int32), (0, pad)).reshape(
        EPAD // STREAM, STREAM)
    rows2d = jnp.pad(adj_rows.astype(jnp.int32), (0, pad)).reshape(
        EPAD // STREAM, STREAM)
    vals1d = jnp.pad(adj_vals, (0, pad))

    ego = jnp.concatenate([user_emb, item_emb], axis=0)
    norms = [ego]
    for k in range(3):
        s = _sc_gather(ego, cols2d, vals1d)
        side = _sc_scatter(s, rows2d)
        ego, nrm = _tc_dense(side, ego, Wg[k], bg[k], Wb[k], bb[k])
        norms.append(nrm)

    ae = jnp.concatenate(norms, axis=1)

    idx = jnp.concatenate([
        users.astype(jnp.int32),
        NU + pos_items.astype(jnp.int32),
        NU + neg_items.astype(jnp.int32),
    ]).reshape(N_IDX // STREAM, STREAM)
    picked = _sc_batch_gather(ae, idx)
    u_out = picked[:NBATCH]
    pos_i = picked[NBATCH:2 * NBATCH]
    neg_i = picked[2 * NBATCH:]

    lp, ln, pp, pn = _tc_logits(u_out, pos_i, neg_i)
    logits = jnp.concatenate([lp, ln], axis=0)
    prediction = jnp.concatenate([pp, pn], axis=0)
    i_sel = jnp.concatenate([pos_i, neg_i], axis=0)

    return (ae, u_out, i_sel, pos_i, neg_i, logits, prediction)
